# Initial kernel scaffold; baseline (speedup 1.0000x reference)
import functools

import jax
import jax.numpy as jnp
from jax import lax
from jax.experimental import pallas as pl
from jax.experimental.pallas import tpu as pltpu
from jax.experimental.pallas import tpu_sc as plsc

P = 7
S = 2
SCALE = 0.0625
B, C, H, W = 2, 256, 64, 64
N = 512
NW = 32
CG = C // NW
NG = N // 16
L = 16


def _splat_i32(x):
    return jnp.full((L,), x, dtype=jnp.int32)


def _sc_body(feat_hbm, rois_hbm, out_hbm, feat_v, rois_v, outbuf_v):
    wid = lax.axis_index("s") * 2 + lax.axis_index("c")
    pltpu.sync_copy(feat_hbm.at[wid], feat_v)
    pltpu.sync_copy(rois_hbm, rois_v)
    lane = lax.iota(jnp.int32, L)
    zero = jnp.zeros((L,), jnp.float32)

    def group_body(g, _):
        sl = pl.ds(g * 16, 16)
        bidx = rois_v[0, sl].astype(jnp.int32)
        x1 = rois_v[1, sl] * SCALE - 0.5
        y1 = rois_v[2, sl] * SCALE - 0.5
        x2 = rois_v[3, sl] * SCALE - 0.5
        y2 = rois_v[4, sl] * SCALE - 0.5
        bin_w = (x2 - x1) * (1.0 / P)
        bin_h = (y2 - y1) * (1.0 / P)
        base = bidx * (H * W)

        def zero_body(j, _):
            c = j // (P * P)
            r = j - c * (P * P)
            ph = r // P
            pw = r - ph * P
            plsc.store_scatter(
                outbuf_v,
                [lane, jnp.full((L,), c, jnp.int32),
                 jnp.full((L,), ph, jnp.int32), jnp.full((L,), pw, jnp.int32)],
                zero)
            return 0
        lax.fori_loop(0, CG * P * P, zero_body, 0)

        def axis_coords(g_idx, start, binsz, limit):
            gv = jnp.full((L,), g_idx, dtype=jnp.int32)
            g2 = gv // 2
            rem = gv - 2 * g2
            grid = g2.astype(jnp.float32) + rem.astype(jnp.float32) * 0.5 + 0.25
            t = start + grid * binsz
            ok = (t > -1.0) & (t < float(limit))
            tc = jnp.clip(t, 0.0, float(limit - 1))
            t0f = jnp.floor(tc)
            lo = tc - t0f
            hi = 1.0 - lo
            lo = jnp.where(ok, lo, 0.0)
            hi = jnp.where(ok, hi, 0.0)
            t0 = t0f.astype(jnp.int32)
            t1 = jnp.minimum(t0 + 1, limit - 1)
            return t0, t1, hi, lo

        def gy_body(gy, _):
            y0, yy1, hy, ly = axis_coords(gy, y1, bin_h, H)
            offy0 = base + y0 * W
            offy1 = base + yy1 * W
            ph = gy // 2

            def gx_body(gx, _):
                x0, xx1, hx, lx = axis_coords(gx, x1, bin_w, W)
                pw = gx // 2
                off00 = offy0 + x0
                off01 = offy0 + xx1
                off10 = offy1 + x0
                off11 = offy1 + xx1
                w00 = hy * hx * 0.25
                w01 = hy * lx * 0.25
                w10 = ly * hx * 0.25
                w11 = ly * lx * 0.25
                phv = jnp.full((L,), ph, dtype=jnp.int32)
                pwv = jnp.full((L,), pw, dtype=jnp.int32)
                for c in range(CG):
                    cs = _splat_i32(c)
                    v = (w00 * plsc.load_gather(feat_v, [off00, cs])
                         + w01 * plsc.load_gather(feat_v, [off01, cs])
                         + w10 * plsc.load_gather(feat_v, [off10, cs])
                         + w11 * plsc.load_gather(feat_v, [off11, cs]))
                    plsc.addupdate_scatter(outbuf_v, [lane, cs, phv, pwv], v)
                return 0
            lax.fori_loop(0, S * P, gx_body, 0)
            return 0
        lax.fori_loop(0, S * P, gy_body, 0)

        pltpu.sync_copy(
            outbuf_v,
            out_hbm.at[pl.ds(g * 16, 16), pl.ds(wid * CG, CG)])
        return 0

    lax.fori_loop(0, NG, group_body, 0)


@jax.jit
def kernel(inputs, rois):
    feat = jnp.transpose(inputs, (0, 2, 3, 1))
    feat = feat.reshape(B, H * W, NW, CG)
    feat = jnp.transpose(feat, (2, 0, 1, 3)).reshape(NW, B * H * W, CG)
    rois_t = rois.T

    mesh = plsc.VectorSubcoreMesh(
        core_axis_name="c", subcore_axis_name="s", num_cores=2,
        num_subcores=16)
    f = pl.kernel(
        _sc_body,
        out_type=jax.ShapeDtypeStruct((N, C, P, P), jnp.float32),
        mesh=mesh,
        scratch_types=[
            pltpu.VMEM((B * H * W, CG), jnp.float32),
            pltpu.VMEM((5, N), jnp.float32),
            pltpu.VMEM((16, CG, P, P), jnp.float32),
        ],
    )
    return f(feat, rois_t)


# speedup vs baseline: 2.7373x; 2.7373x over previous
import jax
import jax.numpy as jnp
from jax import lax
from jax.experimental import pallas as pl
from jax.experimental.pallas import tpu as pltpu
from jax.experimental.pallas import tpu_sc as plsc

P = 7
S = 2
SCALE = 0.0625
B, C, H, W = 2, 256, 64, 64
N = 512
NW = 32
CG = C // NW
NG = N // 16
L = 16
G = P * S


def _axis_coords(g_idx, start, binsz, limit):
    gv = jnp.full((L,), g_idx, dtype=jnp.int32)
    g2 = gv // 2
    rem = gv - 2 * g2
    grid = g2.astype(jnp.float32) + rem.astype(jnp.float32) * 0.5 + 0.25
    t = start + grid * binsz
    ok = (t > -1.0) & (t < float(limit))
    tc = jnp.clip(t, 0.0, float(limit - 1))
    t0 = tc.astype(jnp.int32)
    lo = tc - t0.astype(jnp.float32)
    hi = 1.0 - lo
    lo = jnp.where(ok, lo, 0.0)
    hi = jnp.where(ok, hi, 0.0)
    t1 = jnp.minimum(t0 + 1, limit - 1)
    return t0, t1, hi, lo


def _sc_body(feat_hbm, rois_hbm, out_hbm, feat_v, rois_v, xc_v, outbuf_v):
    wid = lax.axis_index("s") * 2 + lax.axis_index("c")
    pltpu.sync_copy(feat_hbm.at[wid], feat_v)
    pltpu.sync_copy(rois_hbm, rois_v)

    def group_body(g, _):
        sl = pl.ds(g * 16, 16)
        bidx = rois_v[0, sl].astype(jnp.int32)
        x1 = rois_v[1, sl] * SCALE - 0.5
        y1 = rois_v[2, sl] * SCALE - 0.5
        x2 = rois_v[3, sl] * SCALE - 0.5
        y2 = rois_v[4, sl] * SCALE - 0.5
        bin_w = (x2 - x1) * (1.0 / P)
        bin_h = (y2 - y1) * (1.0 / P)
        base = bidx * (H * W)

        def xprep_body(gx, _):
            px0, px1, phx, plx = _axis_coords(gx, x1, bin_w, W)
            xc_v[gx * 4 + 0] = px0
            xc_v[gx * 4 + 1] = px1
            xc_v[gx * 4 + 2] = lax.bitcast_convert_type(phx, jnp.int32)
            xc_v[gx * 4 + 3] = lax.bitcast_convert_type(plx, jnp.int32)
            return 0
        lax.fori_loop(0, G, xprep_body, 0)

        def ph_body(ph, _):
            ya0, ya1, hya, lya = _axis_coords(2 * ph, y1, bin_h, H)
            yb0, yb1, hyb, lyb = _axis_coords(2 * ph + 1, y1, bin_h, H)
            offa0 = base + ya0 * W
            offa1 = base + ya1 * W
            offb0 = base + yb0 * W
            offb1 = base + yb1 * W
            yoffs = ((offa0, offa1, hya, lya), (offb0, offb1, hyb, lyb))

            def pw_body(pw, _):
                xs = []
                for ix in range(2):
                    r = (2 * pw + ix) * 4
                    xs.append((xc_v[r + 0], xc_v[r + 1],
                               lax.bitcast_convert_type(xc_v[r + 2], jnp.float32),
                               lax.bitcast_convert_type(xc_v[r + 3], jnp.float32)))
                offs = []
                wts = []
                for iy in range(2):
                    oy0, oy1, hy, ly = yoffs[iy]
                    for ix in range(2):
                        ox0, ox1, hx, lx = xs[ix]
                        offs += [oy0 + ox0, oy0 + ox1, oy1 + ox0, oy1 + ox1]
                        wts += [hy * hx, hy * lx, ly * hx, ly * lx]
                for c in range(CG):
                    cs = jnp.full((L,), c, dtype=jnp.int32)
                    acc = jnp.zeros((L,), jnp.float32)
                    for k in range(16):
                        acc = acc + wts[k] * plsc.load_gather(
                            feat_v, [offs[k], cs])
                    row = c * (P * P) + ph * P + pw
                    outbuf_v[row] = acc * 0.25
                return 0
            lax.fori_loop(0, P, pw_body, 0)
            return 0
        lax.fori_loop(0, P, ph_body, 0)

        pltpu.sync_copy(
            outbuf_v,
            out_hbm.at[pl.ds(wid * (CG * P * P), CG * P * P),
                       pl.ds(g * 16, 16)])
        return 0

    lax.fori_loop(0, NG, group_body, 0)


@jax.jit
def kernel(inputs, rois):
    feat = jnp.transpose(inputs, (0, 2, 3, 1))
    feat = feat.reshape(B, H * W, NW, CG)
    feat = jnp.transpose(feat, (2, 0, 1, 3)).reshape(NW, B * H * W, CG)
    rois_t = rois.T

    mesh = plsc.VectorSubcoreMesh(
        core_axis_name="c", subcore_axis_name="s", num_cores=2,
        num_subcores=16)
    f = pl.kernel(
        _sc_body,
        out_type=jax.ShapeDtypeStruct((C * P * P, N), jnp.float32),
        mesh=mesh,
        compiler_params=pltpu.CompilerParams(
            use_tc_tiling_on_sc=False, needs_layout_passes=False),
        scratch_types=[
            pltpu.VMEM((B * H * W, CG), jnp.float32),
            pltpu.VMEM((5, N), jnp.float32),
            pltpu.VMEM((G * 4, L), jnp.int32),
            pltpu.VMEM((CG * P * P, L), jnp.float32),
        ],
    )
    out_t = f(feat, rois_t)
    return jnp.transpose(out_t.reshape(C, P, P, N), (3, 0, 1, 2))


# speedup vs baseline: 4.9047x; 1.7918x over previous
import jax
import jax.numpy as jnp
from jax import lax
from jax.experimental import pallas as pl
from jax.experimental.pallas import tpu as pltpu
from jax.experimental.pallas import tpu_sc as plsc

P = 7
S = 2
SCALE = 0.0625
B, C, H, W = 2, 256, 64, 64
N = 512
NW = 32
CG = C // NW
NG = N // 16
L = 16
G = P * S
HW = H * W


def _axis_coords(g_idx, start, binsz, limit):
    gv = jnp.full((L,), g_idx, dtype=jnp.int32)
    g2 = gv // 2
    rem = gv - 2 * g2
    grid = g2.astype(jnp.float32) + rem.astype(jnp.float32) * 0.5 + 0.25
    t = start + grid * binsz
    ok = (t > -1.0) & (t < float(limit))
    tc = jnp.clip(t, 0.0, float(limit - 1))
    t0 = tc.astype(jnp.int32)
    lo = tc - t0.astype(jnp.float32)
    hi = 1.0 - lo
    lo = jnp.where(ok, lo, 0.0)
    hi = jnp.where(ok, hi, 0.0)
    t1 = jnp.minimum(t0 + 1, limit - 1)
    return t0, t1, hi, lo


def _sc_body(in_hbm, rois_hbm, out_hbm, feat_v, rois_v, xc_v, yc_v,
             outbuf_v, sem):
    wid = lax.axis_index("s") * 2 + lax.axis_index("c")
    copies = []
    for b in range(B):
        for c in range(CG):
            copies.append(pltpu.async_copy(
                in_hbm.at[b, wid * CG + c],
                feat_v.at[b * CG + c], sem))
    copies.append(pltpu.async_copy(rois_hbm, rois_v, sem))
    for cp in copies:
        cp.wait()

    lane = lax.iota(jnp.int32, L)

    def group_body(g, _):
        fbase = g * 80 + lane * 5
        bidx = plsc.load_gather(rois_v, [fbase]).astype(jnp.int32)
        x1 = plsc.load_gather(rois_v, [fbase + 1]) * SCALE - 0.5
        y1 = plsc.load_gather(rois_v, [fbase + 2]) * SCALE - 0.5
        x2 = plsc.load_gather(rois_v, [fbase + 3]) * SCALE - 0.5
        y2 = plsc.load_gather(rois_v, [fbase + 4]) * SCALE - 0.5
        bin_w = (x2 - x1) * (1.0 / P)
        bin_h = (y2 - y1) * (1.0 / P)
        bplane = bidx * CG

        @plsc.parallel_loop(0, G, 1, unroll=2)
        def xprep_body(gx):
            px0, px1, phx, plx = _axis_coords(gx, x1, bin_w, W)
            xc_v[gx * 4 + 0] = px0
            xc_v[gx * 4 + 1] = px1
            xc_v[gx * 4 + 2] = lax.bitcast_convert_type(phx, jnp.int32)
            xc_v[gx * 4 + 3] = lax.bitcast_convert_type(plx, jnp.int32)

        @plsc.parallel_loop(0, G, 1, unroll=2)
        def yprep_body(gy):
            py0, py1, phy, ply = _axis_coords(gy, y1, bin_h, H)
            yc_v[gy * 4 + 0] = py0 * W
            yc_v[gy * 4 + 1] = py1 * W
            yc_v[gy * 4 + 2] = lax.bitcast_convert_type(phy * 0.25, jnp.int32)
            yc_v[gy * 4 + 3] = lax.bitcast_convert_type(ply * 0.25, jnp.int32)

        @plsc.parallel_loop(0, P * P, 1, unroll=2)
        def bin_body(bi):
            ph = bi // P
            pw = bi - ph * P
            ys = []
            xs = []
            for i in range(2):
                ry = (2 * ph + i) * 4
                ys.append((yc_v[ry + 0], yc_v[ry + 1],
                           lax.bitcast_convert_type(yc_v[ry + 2], jnp.float32),
                           lax.bitcast_convert_type(yc_v[ry + 3], jnp.float32)))
                rx = (2 * pw + i) * 4
                xs.append((xc_v[rx + 0], xc_v[rx + 1],
                           lax.bitcast_convert_type(xc_v[rx + 2], jnp.float32),
                           lax.bitcast_convert_type(xc_v[rx + 3], jnp.float32)))
            offs = []
            wts = []
            for iy in range(2):
                oy0, oy1, hy, ly = ys[iy]
                for ix in range(2):
                    ox0, ox1, hx, lx = xs[ix]
                    offs += [oy0 + ox0, oy0 + ox1, oy1 + ox0, oy1 + ox1]
                    wts += [hy * hx, hy * lx, ly * hx, ly * lx]
            for c in range(CG):
                pv = bplane + c
                p = [wts[k] * plsc.load_gather(feat_v, [pv, offs[k]])
                     for k in range(16)]
                while len(p) > 1:
                    p = [p[i] + p[i + 1] for i in range(0, len(p), 2)]
                outbuf_v[c * (P * P) + bi] = p[0]

        pltpu.sync_copy(
            outbuf_v,
            out_hbm.at[pl.ds(wid * (CG * P * P), CG * P * P),
                       pl.ds(g * 16, 16)])
        return 0

    lax.fori_loop(0, NG, group_body, 0)


@jax.jit
def kernel(inputs, rois):
    mesh = plsc.VectorSubcoreMesh(
        core_axis_name="c", subcore_axis_name="s", num_cores=2,
        num_subcores=16)
    f = pl.kernel(
        _sc_body,
        out_type=jax.ShapeDtypeStruct((C * P * P, N), jnp.float32),
        mesh=mesh,
        compiler_params=pltpu.CompilerParams(
            use_tc_tiling_on_sc=False, needs_layout_passes=False),
        scratch_types=[
            pltpu.VMEM((B * CG, HW), jnp.float32),
            pltpu.VMEM((N * 5,), jnp.float32),
            pltpu.VMEM((G * 4, L), jnp.int32),
            pltpu.VMEM((G * 4, L), jnp.int32),
            pltpu.VMEM((CG * P * P, L), jnp.float32),
            pltpu.SemaphoreType.DMA,
        ],
    )
    out_t = f(inputs.reshape(B, C, HW), rois.reshape(N * 5))
    return jnp.transpose(out_t.reshape(C, P, P, N), (3, 0, 1, 2))


# speedup vs baseline: 6.2576x; 1.2759x over previous
import jax
import jax.numpy as jnp
from jax import lax
from jax.experimental import pallas as pl
from jax.experimental.pallas import tpu as pltpu
from jax.experimental.pallas import tpu_sc as plsc

P = 7
S = 2
SCALE = 0.0625
B, C, H, W = 2, 256, 64, 64
N = 512
NW = 32
CG = C // NW
NG = N // 16
L = 16
G = P * S
HW = H * W


def _axis_coords(g_idx, start, binsz, limit):
    gv = jnp.full((L,), g_idx, dtype=jnp.int32)
    g2 = gv // 2
    rem = gv - 2 * g2
    grid = g2.astype(jnp.float32) + rem.astype(jnp.float32) * 0.5 + 0.25
    t = start + grid * binsz
    ok = (t > -1.0) & (t < float(limit))
    tc = jnp.clip(t, 0.0, float(limit - 1))
    t0 = tc.astype(jnp.int32)
    lo = tc - t0.astype(jnp.float32)
    hi = 1.0 - lo
    lo = jnp.where(ok, lo, 0.0)
    hi = jnp.where(ok, hi, 0.0)
    t1 = jnp.minimum(t0 + 1, limit - 1)
    return t0, t1, hi, lo


def _sc_body(in_hbm, rois_hbm, out_hbm, feat_v, rois_v, xc_v, yc_v,
             outbuf_v, sem, osem):
    wid = lax.axis_index("s") * 2 + lax.axis_index("c")
    copies = [
        pltpu.async_copy(
            in_hbm.at[pl.ds((b * C + wid * CG) * HW, CG * HW)],
            feat_v.at[pl.ds(b * CG * HW, CG * HW)], sem)
        for b in range(B)
    ]
    copies.append(pltpu.async_copy(rois_hbm, rois_v, sem))
    for cp in copies:
        cp.wait()

    lane = lax.iota(jnp.int32, L)

    def group_body(g, _):
        par = g & 1
        @pl.when(g >= 2)
        def _wait_prev():
            pltpu.make_async_copy(
                outbuf_v.at[par],
                out_hbm.at[pl.ds(wid * (CG * P * P), CG * P * P),
                           pl.ds((g - 2) * 16, 16)],
                osem).wait()
        fbase = g * 80 + lane * 5
        bidx = plsc.load_gather(rois_v, [fbase]).astype(jnp.int32)
        x1 = plsc.load_gather(rois_v, [fbase + 1]) * SCALE - 0.5
        y1 = plsc.load_gather(rois_v, [fbase + 2]) * SCALE - 0.5
        x2 = plsc.load_gather(rois_v, [fbase + 3]) * SCALE - 0.5
        y2 = plsc.load_gather(rois_v, [fbase + 4]) * SCALE - 0.5
        bin_w = (x2 - x1) * (1.0 / P)
        bin_h = (y2 - y1) * (1.0 / P)
        bbase = bidx * (CG * HW)

        @plsc.parallel_loop(0, G, 1, unroll=2)
        def xprep_body(gx):
            px0, px1, phx, plx = _axis_coords(gx, x1, bin_w, W)
            xc_v[gx * 4 + 0] = px0
            xc_v[gx * 4 + 1] = px1
            xc_v[gx * 4 + 2] = lax.bitcast_convert_type(phx, jnp.int32)
            xc_v[gx * 4 + 3] = lax.bitcast_convert_type(plx, jnp.int32)

        @plsc.parallel_loop(0, G, 1, unroll=2)
        def yprep_body(gy):
            py0, py1, phy, ply = _axis_coords(gy, y1, bin_h, H)
            yc_v[gy * 4 + 0] = bbase + py0 * W
            yc_v[gy * 4 + 1] = bbase + py1 * W
            yc_v[gy * 4 + 2] = lax.bitcast_convert_type(phy * 0.25, jnp.int32)
            yc_v[gy * 4 + 3] = lax.bitcast_convert_type(ply * 0.25, jnp.int32)

        @plsc.parallel_loop(0, P * P, 1, unroll=2)
        def bin_body(bi):
            ph = bi // P
            pw = bi - ph * P
            ys = []
            xs = []
            for i in range(2):
                ry = (2 * ph + i) * 4
                ys.append((yc_v[ry + 0], yc_v[ry + 1],
                           lax.bitcast_convert_type(yc_v[ry + 2], jnp.float32),
                           lax.bitcast_convert_type(yc_v[ry + 3], jnp.float32)))
                rx = (2 * pw + i) * 4
                xs.append((xc_v[rx + 0], xc_v[rx + 1],
                           lax.bitcast_convert_type(xc_v[rx + 2], jnp.float32),
                           lax.bitcast_convert_type(xc_v[rx + 3], jnp.float32)))
            offs = []
            wts = []
            for iy in range(2):
                oy0, oy1, hy, ly = ys[iy]
                for ix in range(2):
                    ox0, ox1, hx, lx = xs[ix]
                    offs += [oy0 + ox0, oy0 + ox1, oy1 + ox0, oy1 + ox1]
                    wts += [hy * hx, hy * lx, ly * hx, ly * lx]
            for c in range(CG):
                cofs = jnp.full((L,), c * HW, dtype=jnp.int32)
                p = [wts[k] * plsc.load_gather(feat_v, [offs[k] + cofs])
                     for k in range(16)]
                while len(p) > 1:
                    p = [p[i] + p[i + 1] for i in range(0, len(p), 2)]
                outbuf_v[par, c * (P * P) + bi] = p[0]

        pltpu.async_copy(
            outbuf_v.at[par],
            out_hbm.at[pl.ds(wid * (CG * P * P), CG * P * P),
                       pl.ds(g * 16, 16)], osem)
        return 0

    lax.fori_loop(0, NG, group_body, 0)
    for g in (NG - 2, NG - 1):
        pltpu.make_async_copy(
            outbuf_v.at[g & 1],
            out_hbm.at[pl.ds(wid * (CG * P * P), CG * P * P),
                       pl.ds(g * 16, 16)],
            osem).wait()


@jax.jit
def kernel(inputs, rois):
    mesh = plsc.VectorSubcoreMesh(
        core_axis_name="c", subcore_axis_name="s", num_cores=2,
        num_subcores=16)
    f = pl.kernel(
        _sc_body,
        out_type=jax.ShapeDtypeStruct((C * P * P, N), jnp.float32),
        mesh=mesh,
        compiler_params=pltpu.CompilerParams(
            use_tc_tiling_on_sc=False, needs_layout_passes=False),
        scratch_types=[
            pltpu.VMEM((B * CG * HW,), jnp.float32),
            pltpu.VMEM((N * 5,), jnp.float32),
            pltpu.VMEM((G * 4, L), jnp.int32),
            pltpu.VMEM((G * 4, L), jnp.int32),
            pltpu.VMEM((2, CG * P * P, L), jnp.float32),
            pltpu.SemaphoreType.DMA,
            pltpu.SemaphoreType.DMA,
        ],
    )
    out_t = f(inputs.reshape(B * C * HW), rois.reshape(N * 5))
    return jnp.transpose(out_t.reshape(C, P, P, N), (3, 0, 1, 2))
